# baseline (speedup 1.0000x reference)
import jax
import jax.numpy as jnp
from jax.experimental import pallas as pl
from jax.experimental.pallas import tpu as pltpu

B = 1024
K = 40960
KBLK = 2560
NK = K // KBLK


def _body(wfts_ref, bfts_ref, stm_ref, Wft_ref, bft_ref, W1_ref, b1_ref,
          W2_ref, b2_ref, out_ref, accw_ref, accb_ref):
    k = pl.program_id(0)

    @pl.when(k == 0)
    def _():
        accw_ref[...] = jnp.zeros_like(accw_ref)
        accb_ref[...] = jnp.zeros_like(accb_ref)

    dn = (((1,), (1,)), ((), ()))
    wslice = Wft_ref[:, pl.ds(k * KBLK, KBLK)]
    accw_ref[...] += jax.lax.dot_general(
        wslice, wfts_ref[...], dn, preferred_element_type=jnp.float32)
    accb_ref[...] += jax.lax.dot_general(
        wslice, bfts_ref[...], dn, preferred_element_type=jnp.float32)

    @pl.when(k == NK - 1)
    def _():
        w = accw_ref[...] + bft_ref[...]
        b = accb_ref[...] + bft_ref[...]
        stm = stm_ref[...]
        cat_wb = jnp.concatenate([w, b], axis=0)
        cat_bw = jnp.concatenate([b, w], axis=0)
        acc = stm * cat_wb + (1.0 - stm) * cat_bw
        x1 = jnp.clip(acc, 0.0, 1.0)
        h = jax.lax.dot_general(W1_ref[...], x1, (((1,), (0,)), ((), ())),
                                preferred_element_type=jnp.float32)
        h = jnp.clip(h + b1_ref[...], 0.0, 1.0)
        out = jax.lax.dot_general(W2_ref[...], h, (((1,), (0,)), ((), ())),
                                  preferred_element_type=jnp.float32)
        out_ref[...] = out + b2_ref[0]


def kernel(wfts, bfts, stm, W_ft, b_ft, W1, b1, W2, b2):
    bftbc = jnp.broadcast_to(b_ft.reshape(4, 1), (4, B))
    b1bc = jnp.broadcast_to(b1.reshape(8, 1), (8, B))
    W2p = jnp.zeros((8, 8), jnp.float32).at[0, :].set(W2[0])
    out = pl.pallas_call(
        _body,
        grid=(NK,),
        in_specs=[
            pl.BlockSpec((B, KBLK), lambda k: (0, k)),
            pl.BlockSpec((B, KBLK), lambda k: (0, k)),
            pl.BlockSpec((1, B), lambda k: (0, 0)),
            pl.BlockSpec((4, K), lambda k: (0, 0)),
            pl.BlockSpec((4, B), lambda k: (0, 0)),
            pl.BlockSpec((8, 8), lambda k: (0, 0)),
            pl.BlockSpec((8, B), lambda k: (0, 0)),
            pl.BlockSpec((8, 8), lambda k: (0, 0)),
            pl.BlockSpec(memory_space=pltpu.SMEM),
        ],
        out_specs=pl.BlockSpec((8, B), lambda k: (0, 0)),
        out_shape=jax.ShapeDtypeStruct((8, B), jnp.float32),
        scratch_shapes=[
            pltpu.VMEM((4, B), jnp.float32),
            pltpu.VMEM((4, B), jnp.float32),
        ],
    )(wfts, bfts, stm.reshape(1, B), W_ft, bftbc, W1, b1bc, W2p, b2)
    return out[0].reshape(B, 1)


# speedup vs baseline: 1.0077x; 1.0077x over previous
import jax
import jax.numpy as jnp
from jax.experimental import pallas as pl
from jax.experimental.pallas import tpu as pltpu

B = 1024
K = 40960
KBLK = 2048
NK = K // KBLK


def _body(wfts_ref, bfts_ref, stm_ref, Wft_ref, bft_ref, W1_ref, b1_ref,
          W2_ref, b2_ref, out_ref, accw_ref, accb_ref):
    k = pl.program_id(0)

    @pl.when(k == 0)
    def _():
        accw_ref[...] = jnp.zeros_like(accw_ref)
        accb_ref[...] = jnp.zeros_like(accb_ref)

    dn = (((1,), (1,)), ((), ()))
    wslice = Wft_ref[:, pl.ds(k * KBLK, KBLK)]
    accw_ref[...] += jax.lax.dot_general(
        wslice, wfts_ref[...], dn, preferred_element_type=jnp.float32)
    accb_ref[...] += jax.lax.dot_general(
        wslice, bfts_ref[...], dn, preferred_element_type=jnp.float32)

    @pl.when(k == NK - 1)
    def _():
        w = accw_ref[...] + bft_ref[...]
        b = accb_ref[...] + bft_ref[...]
        stm = stm_ref[...]
        cat_wb = jnp.concatenate([w, b], axis=0)
        cat_bw = jnp.concatenate([b, w], axis=0)
        acc = stm * cat_wb + (1.0 - stm) * cat_bw
        x1 = jnp.clip(acc, 0.0, 1.0)
        h = jax.lax.dot_general(W1_ref[...], x1, (((1,), (0,)), ((), ())),
                                preferred_element_type=jnp.float32)
        h = jnp.clip(h + b1_ref[...], 0.0, 1.0)
        out = jax.lax.dot_general(W2_ref[...], h, (((1,), (0,)), ((), ())),
                                  preferred_element_type=jnp.float32)
        out_ref[...] = out + b2_ref[0]


def kernel(wfts, bfts, stm, W_ft, b_ft, W1, b1, W2, b2):
    bftbc = jnp.broadcast_to(b_ft.reshape(4, 1), (4, B))
    b1bc = jnp.broadcast_to(b1.reshape(8, 1), (8, B))
    W2p = jnp.zeros((8, 8), jnp.float32).at[0, :].set(W2[0])
    out = pl.pallas_call(
        _body,
        grid=(NK,),
        in_specs=[
            pl.BlockSpec((B, KBLK), lambda k: (0, k)),
            pl.BlockSpec((B, KBLK), lambda k: (0, k)),
            pl.BlockSpec((1, B), lambda k: (0, 0)),
            pl.BlockSpec((4, K), lambda k: (0, 0)),
            pl.BlockSpec((4, B), lambda k: (0, 0)),
            pl.BlockSpec((8, 8), lambda k: (0, 0)),
            pl.BlockSpec((8, B), lambda k: (0, 0)),
            pl.BlockSpec((8, 8), lambda k: (0, 0)),
            pl.BlockSpec(memory_space=pltpu.SMEM),
        ],
        out_specs=pl.BlockSpec((8, B), lambda k: (0, 0)),
        out_shape=jax.ShapeDtypeStruct((8, B), jnp.float32),
        scratch_shapes=[
            pltpu.VMEM((4, B), jnp.float32),
            pltpu.VMEM((4, B), jnp.float32),
        ],
    )(wfts, bfts, stm.reshape(1, B), W_ft, bftbc, W1, b1bc, W2p, b2)
    return out[0].reshape(B, 1)
